# baseline (speedup 1.0000x reference)
import functools

import jax
import jax.numpy as jnp
from jax import lax
from jax.experimental import pallas as pl
from jax.experimental.pallas import tpu as pltpu
from jax.experimental.pallas import tpu_sc as plsc

N = 10000
H = 8
NC = 2
NS = 16
LANES = 16
B = 64
HALF = N // 2
ZROWS = 320
SPROWS = 5008
TRASH = HALF
EXP_ROWS = 320
AW = 128


def _leaky(x, slope):
    return jnp.where(x < 0, x * slope, x)


ROWB = 1000
GRID = N // ROWB


def _proj_body(x_ref, w_ref, as_ref, ad_ref, xt_ref, a_ref, *, hc):
    c = hc // H
    xt = jnp.dot(x_ref[...], w_ref[...], preferred_element_type=jnp.float32)
    xt_ref[...] = xt
    xr = xt.reshape(ROWB, H, c)
    a_s = jnp.sum(xr * as_ref[...][None], axis=-1)
    a_d = jnp.sum(xr * ad_ref[...][None], axis=-1)
    pad = jnp.zeros((ROWB, AW - 2 * H), jnp.float32)
    a_ref[...] = jnp.concatenate([a_s, a_d, pad], axis=1)


def _proj(x, w, att_s, att_d):
    d = x.shape[1]
    hc = w.shape[1]
    return pl.pallas_call(
        functools.partial(_proj_body, hc=hc),
        grid=(GRID,),
        in_specs=[
            pl.BlockSpec((ROWB, d), lambda i: (i, 0)),
            pl.BlockSpec((d, hc), lambda i: (0, 0)),
            pl.BlockSpec(att_s.shape, lambda i: (0, 0)),
            pl.BlockSpec(att_d.shape, lambda i: (0, 0)),
        ],
        out_specs=[
            pl.BlockSpec((ROWB, hc), lambda i: (i, 0)),
            pl.BlockSpec((ROWB, AW), lambda i: (i, 0)),
        ],
        out_shape=[
            jax.ShapeDtypeStruct((N, hc), jnp.float32),
            jax.ShapeDtypeStruct((N, AW), jnp.float32),
        ],
    )(x, w, att_s, att_d)


def _norm_block(num, den, b):
    hc = num.shape[1]
    c = hc // H
    d8 = den[:, :H]
    drep = jnp.broadcast_to(d8[:, :, None], (ROWB, H, c)).reshape(ROWB, hc)
    return _leaky(num / (drep + 1e-16) + b, 0.01)


def _proj2_body(num_ref, den_ref, b_ref, w_ref, as_ref, ad_ref, xt_ref, a_ref,
                *, hc):
    c = hc // H
    h = _norm_block(num_ref[...], den_ref[...], b_ref[...])
    xt = jnp.dot(h, w_ref[...], preferred_element_type=jnp.float32)
    xt_ref[...] = xt
    xr = xt.reshape(ROWB, H, c)
    a_s = jnp.sum(xr * as_ref[...][None], axis=-1)
    a_d = jnp.sum(xr * ad_ref[...][None], axis=-1)
    pad = jnp.zeros((ROWB, AW - 2 * H), jnp.float32)
    a_ref[...] = jnp.concatenate([a_s, a_d, pad], axis=1)


def _proj2(num, den, b, w, att_s, att_d):
    d = num.shape[1]
    hc = w.shape[1]
    return pl.pallas_call(
        functools.partial(_proj2_body, hc=hc),
        grid=(GRID,),
        in_specs=[
            pl.BlockSpec((ROWB, d), lambda i: (i, 0)),
            pl.BlockSpec((ROWB, 2 * H), lambda i: (i, 0)),
            pl.BlockSpec((1, d), lambda i: (0, 0)),
            pl.BlockSpec((d, hc), lambda i: (0, 0)),
            pl.BlockSpec(att_s.shape, lambda i: (0, 0)),
            pl.BlockSpec(att_d.shape, lambda i: (0, 0)),
        ],
        out_specs=[
            pl.BlockSpec((ROWB, hc), lambda i: (i, 0)),
            pl.BlockSpec((ROWB, AW), lambda i: (i, 0)),
        ],
        out_shape=[
            jax.ShapeDtypeStruct((N, hc), jnp.float32),
            jax.ShapeDtypeStruct((N, AW), jnp.float32),
        ],
    )(num, den, b, w, att_s, att_d)


def _head_body(num_ref, den_ref, b_ref, fcw_ref, fcb_ref,
               nreps_ref, grep_ref, probs_ref, acc_ref):
    i = pl.program_id(0)
    nb = _norm_block(num_ref[...], den_ref[...], b_ref[...])
    nreps_ref[...] = nb

    @pl.when(i == 0)
    def _():
        acc_ref[...] = jnp.zeros_like(acc_ref)

    acc_ref[...] += jnp.sum(nb, axis=0, keepdims=True)

    @pl.when(i == GRID - 1)
    def _():
        g = acc_ref[...] / jnp.float32(N)
        grep_ref[...] = g
        lg = jnp.dot(g, fcw_ref[...], preferred_element_type=jnp.float32)
        lg = lg + fcb_ref[...]
        m = jnp.max(lg, axis=1, keepdims=True)
        e = jnp.exp(lg - m)
        probs_ref[...] = e / jnp.sum(e, axis=1, keepdims=True)


def _head(num, den, b, fcw, fcb):
    hc = num.shape[1]
    ncls = fcw.shape[1]
    return pl.pallas_call(
        _head_body,
        grid=(GRID,),
        in_specs=[
            pl.BlockSpec((ROWB, hc), lambda i: (i, 0)),
            pl.BlockSpec((ROWB, 2 * H), lambda i: (i, 0)),
            pl.BlockSpec((1, hc), lambda i: (0, 0)),
            pl.BlockSpec((hc, ncls), lambda i: (0, 0)),
            pl.BlockSpec((1, ncls), lambda i: (0, 0)),
        ],
        out_specs=[
            pl.BlockSpec((ROWB, hc), lambda i: (i, 0)),
            pl.BlockSpec((1, hc), lambda i: (0, 0)),
            pl.BlockSpec((1, ncls), lambda i: (0, 0)),
        ],
        out_shape=[
            jax.ShapeDtypeStruct((N, hc), jnp.float32),
            jax.ShapeDtypeStruct((1, hc), jnp.float32),
            jax.ShapeDtypeStruct((1, ncls), jnp.float32),
        ],
        scratch_shapes=[pltpu.VMEM((1, hc), jnp.float32)],
    )(num, den, b, fcw, fcb)


G = 32
PEND = 96
SCAN = 1024
OWNB = 256
OWN1 = 56
LAST1 = N - NC * NS * OWNB - (NC * NS - 1) * OWN1


def _sc_edge_body(xt, a_nh, src, dst, num_o, den_o,
                  num_l, den_l, grows, asg, adg, aeb, srcv, dstv,
                  spend, dpend, sem, sem2, *, hc, ep):
    core = lax.axis_index("c")
    sub = lax.axis_index("s")
    wid = core * NS + sub
    nchunk = ep // SCAN
    cols_per_head = hc // H
    chunks_per_head = cols_per_head // LANES
    iota = lax.iota(jnp.int32, LANES)
    zf = jnp.zeros((LANES,), jnp.float32)
    zi = jnp.zeros((LANES,), jnp.int32)

    for i in range(G):
        aeb[i, :] = zf
    for g in range(PEND // LANES):
        plsc.store_scatter(spend, [iota + g * LANES], zi)
        plsc.store_scatter(dpend, [iota + g * LANES], zi)

    def compute_ae():
        for g in range(G // LANES):
            ii = iota + (g * LANES)
            for h in range(H):
                hv = jnp.full((LANES,), h, jnp.int32)
                av = plsc.load_gather(asg, [ii, hv])
                dv = plsc.load_gather(adg, [ii, jnp.full((LANES,), h + H,
                                                         jnp.int32)])
                e = jnp.exp(_leaky(av + dv, 0.2))
                plsc.store_scatter(aeb, [ii, hv], e)

    def run_pass(base, ownb, lastb):
        t0 = base + wid * ownb
        own = jnp.where(wid == NC * NS - 1, lastb, ownb)
        zmax = max(ownb, lastb)

        def zrow(r, carry):
            rf = jnp.full((LANES,), r, jnp.int32)
            for cch in range(hc // LANES):
                plsc.store_scatter(num_l, [rf, iota + cch * LANES], zf)
            plsc.store_scatter(den_l, [rf, iota], zf)
            return carry

        lax.fori_loop(0, zmax, zrow, 0)

        def accum_edge(i, carry):
            ifull = jnp.full((LANES,), i, jnp.int32)
            rr = plsc.load_gather(dpend, [ifull]) - t0
            aev = plsc.load_gather(aeb, [ifull, iota])
            plsc.addupdate_scatter(den_l, [rr, iota], aev)
            for h in range(H):
                m = plsc.load_gather(aeb, [ifull, jnp.full((LANES,), h,
                                                           jnp.int32)])
                for q in range(chunks_per_head):
                    col = iota + (h * cols_per_head + q * LANES)
                    v = plsc.load_gather(grows, [ifull, col])
                    plsc.addupdate_scatter(num_l, [rr, col], v * m)
            return carry

        def start_gather():
            pltpu.async_copy(xt.at[spend.at[pl.ds(0, G)]], grows, sem)
            pltpu.async_copy(a_nh.at[spend.at[pl.ds(0, G)]], asg, sem2)
            pltpu.async_copy(a_nh.at[dpend.at[pl.ds(0, G)]], adg, sem2)

        def wait_gather():
            pltpu.make_async_copy(xt.at[spend.at[pl.ds(0, G)]], grows,
                                  sem).wait()
            pltpu.make_async_copy(a_nh.at[spend.at[pl.ds(0, G)]], asg,
                                  sem2).wait()
            pltpu.make_async_copy(a_nh.at[dpend.at[pl.ds(0, G)]], adg,
                                  sem2).wait()

        def drain(nedges, off2):
            wait_gather()
            compute_ae()
            lax.fori_loop(0, nedges, accum_edge, 0)
            for gg in range((G + 15 + LANES - 1) // LANES):
                mvi = iota + (G + gg * LANES)
                mv = mvi < off2
                svr = plsc.load_gather(spend, [mvi])
                dvr = plsc.load_gather(dpend, [mvi])
                plsc.store_scatter(spend, [iota + gg * LANES], svr, mask=mv)
                plsc.store_scatter(dpend, [iota + gg * LANES], dvr, mask=mv)

        def group(gi, carry):
            off, armed = carry
            ii = iota + gi * LANES
            sv = plsc.load_gather(srcv, [ii])
            dv = plsc.load_gather(dstv, [ii])
            r = dv - t0
            ok = (r >= 0) & (r < own)
            csum = plsc.cumsum(jnp.where(ok, 1, 0).astype(jnp.int32))
            pos = off + csum - 1
            plsc.store_scatter(spend, [pos], sv, mask=ok)
            plsc.store_scatter(dpend, [pos], dv, mask=ok)
            off2 = off + jnp.max(csum)

            fire_c = (armed == 0) & (off2 >= G)

            @pl.when(fire_c)
            def _():
                start_gather()

            acc_c = (armed == 1) & (off2 >= 2 * G)

            @pl.when(acc_c)
            def _():
                drain(G, off2)

                @pl.when(off2 - G >= G)
                def _():
                    start_gather()

            off3 = jnp.where(acc_c, off2 - G, off2)
            armed3 = jnp.where(
                acc_c, (off3 >= G).astype(jnp.int32),
                jnp.where(fire_c, 1, armed))
            return (off3, armed3)

        def chunk(ci, carry):
            b0 = ci * SCAN
            pltpu.sync_copy(src.at[pl.ds(b0, SCAN)], srcv)
            pltpu.sync_copy(dst.at[pl.ds(b0, SCAN)], dstv)
            return lax.fori_loop(0, SCAN // LANES, group, carry)

        off, armed = lax.fori_loop(0, nchunk, chunk,
                                   (jnp.int32(0), jnp.int32(0)))

        @pl.when(armed == 1)
        def _():
            drain(G, off)

        offt = jnp.where(armed == 1, off - G, off)

        @pl.when(offt > 0)
        def _():
            start_gather()
            wait_gather()
            compute_ae()
            lax.fori_loop(0, offt, accum_edge, 0)

        @pl.when(wid < NC * NS - 1)
        def _():
            pltpu.sync_copy(num_l.at[pl.ds(0, ownb)],
                            num_o.at[pl.ds(t0, ownb)])
            pltpu.sync_copy(den_l.at[pl.ds(0, ownb)],
                            den_o.at[pl.ds(t0, ownb)])

        @pl.when(wid == NC * NS - 1)
        def _():
            pltpu.sync_copy(num_l.at[pl.ds(0, lastb)],
                            num_o.at[pl.ds(t0, lastb)])
            pltpu.sync_copy(den_l.at[pl.ds(0, lastb)],
                            den_o.at[pl.ds(t0, lastb)])

    run_pass(0, OWNB, OWNB)
    run_pass(NC * NS * OWNB, OWN1, LAST1)


def _sc_edge(xt, a_nh, src, dst, hc, ep):
    mesh = plsc.VectorSubcoreMesh(core_axis_name="c", subcore_axis_name="s",
                                  num_cores=NC, num_subcores=NS)
    kfn = pl.kernel(
        functools.partial(_sc_edge_body, hc=hc, ep=ep),
        out_type=(
            jax.ShapeDtypeStruct((N, hc), jnp.float32),
            jax.ShapeDtypeStruct((N, 2 * H), jnp.float32),
        ),
        mesh=mesh,
        scratch_types=[
            pltpu.VMEM((OWNB, hc), jnp.float32),
            pltpu.VMEM((OWNB, 2 * H), jnp.float32),
            pltpu.VMEM((G, hc), jnp.float32),
            pltpu.VMEM((G, AW), jnp.float32),
            pltpu.VMEM((G, AW), jnp.float32),
            pltpu.VMEM((G, 2 * H), jnp.float32),
            pltpu.VMEM((SCAN,), jnp.int32),
            pltpu.VMEM((SCAN,), jnp.int32),
            pltpu.VMEM((PEND,), jnp.int32),
            pltpu.VMEM((PEND,), jnp.int32),
            pltpu.SemaphoreType.DMA,
            pltpu.SemaphoreType.DMA,
        ],
        compiler_params=pltpu.CompilerParams(needs_layout_passes=False),
    )
    return kfn(xt, a_nh, src, dst)


def kernel(x, edge_index, W1, att_src1, att_dst1, b1,
           W2, att_src2, att_dst2, b2, fcW, fcb):
    e = edge_index.shape[1]
    etot = e + N
    ep = ((etot + SCAN - 1) // SCAN) * SCAN
    pad = ep - etot
    loop = jnp.arange(N, dtype=edge_index.dtype)
    src = jnp.concatenate(
        [edge_index[0], loop, jnp.zeros((pad,), edge_index.dtype)])
    dst = jnp.concatenate(
        [edge_index[1], loop, jnp.full((pad,), -1, edge_index.dtype)])

    xt1, a1 = _proj(x, W1, att_src1, att_dst1)
    num1, den1 = _sc_edge(xt1, a1, src, dst, W1.shape[1], ep)
    xt2, a2 = _proj2(num1, den1, b1.reshape(1, -1), W2, att_src2, att_dst2)
    num2, den2 = _sc_edge(xt2, a2, src, dst, W2.shape[1], ep)
    n_reps, g_rep, probs = _head(num2, den2, b2.reshape(1, -1),
                                 fcW, fcb.reshape(1, -1))
    return (probs, n_reps, g_rep)


# speedup vs baseline: 1.3080x; 1.3080x over previous
import functools

import jax
import jax.numpy as jnp
from jax import lax
from jax.experimental import pallas as pl
from jax.experimental.pallas import tpu as pltpu
from jax.experimental.pallas import tpu_sc as plsc

N = 10000
H = 8
NC = 2
NS = 16
LANES = 16
B = 64
HALF = N // 2
ZROWS = 320
SPROWS = 5008
TRASH = HALF
EXP_ROWS = 320
AW = 128


def _leaky(x, slope):
    return jnp.where(x < 0, x * slope, x)


ROWB = 1000
GRID = N // ROWB


def _proj_body(x_ref, w_ref, as_ref, ad_ref, xt_ref, a_ref, *, hc):
    c = hc // H
    xt = jnp.dot(x_ref[...], w_ref[...], preferred_element_type=jnp.float32)
    xt_ref[...] = xt
    xr = xt.reshape(ROWB, H, c)
    a_s = jnp.sum(xr * as_ref[...][None], axis=-1)
    a_d = jnp.sum(xr * ad_ref[...][None], axis=-1)
    pad = jnp.zeros((ROWB, AW - 2 * H), jnp.float32)
    a_ref[...] = jnp.concatenate([a_s, a_d, pad], axis=1)


def _proj(x, w, att_s, att_d):
    d = x.shape[1]
    hc = w.shape[1]
    return pl.pallas_call(
        functools.partial(_proj_body, hc=hc),
        grid=(GRID,),
        in_specs=[
            pl.BlockSpec((ROWB, d), lambda i: (i, 0)),
            pl.BlockSpec((d, hc), lambda i: (0, 0)),
            pl.BlockSpec(att_s.shape, lambda i: (0, 0)),
            pl.BlockSpec(att_d.shape, lambda i: (0, 0)),
        ],
        out_specs=[
            pl.BlockSpec((ROWB, hc), lambda i: (i, 0)),
            pl.BlockSpec((ROWB, AW), lambda i: (i, 0)),
        ],
        out_shape=[
            jax.ShapeDtypeStruct((N, hc), jnp.float32),
            jax.ShapeDtypeStruct((N, AW), jnp.float32),
        ],
    )(x, w, att_s, att_d)


def _norm_block(num, den, b):
    hc = num.shape[1]
    c = hc // H
    d8 = den[:, :H]
    drep = jnp.broadcast_to(d8[:, :, None], (ROWB, H, c)).reshape(ROWB, hc)
    return _leaky(num / (drep + 1e-16) + b, 0.01)


def _proj2_body(num_ref, den_ref, b_ref, w_ref, as_ref, ad_ref, xt_ref, a_ref,
                *, hc):
    c = hc // H
    h = _norm_block(num_ref[...], den_ref[...], b_ref[...])
    xt = jnp.dot(h, w_ref[...], preferred_element_type=jnp.float32)
    xt_ref[...] = xt
    xr = xt.reshape(ROWB, H, c)
    a_s = jnp.sum(xr * as_ref[...][None], axis=-1)
    a_d = jnp.sum(xr * ad_ref[...][None], axis=-1)
    pad = jnp.zeros((ROWB, AW - 2 * H), jnp.float32)
    a_ref[...] = jnp.concatenate([a_s, a_d, pad], axis=1)


def _proj2(num, den, b, w, att_s, att_d):
    d = num.shape[1]
    hc = w.shape[1]
    return pl.pallas_call(
        functools.partial(_proj2_body, hc=hc),
        grid=(GRID,),
        in_specs=[
            pl.BlockSpec((ROWB, d), lambda i: (i, 0)),
            pl.BlockSpec((ROWB, 2 * H), lambda i: (i, 0)),
            pl.BlockSpec((1, d), lambda i: (0, 0)),
            pl.BlockSpec((d, hc), lambda i: (0, 0)),
            pl.BlockSpec(att_s.shape, lambda i: (0, 0)),
            pl.BlockSpec(att_d.shape, lambda i: (0, 0)),
        ],
        out_specs=[
            pl.BlockSpec((ROWB, hc), lambda i: (i, 0)),
            pl.BlockSpec((ROWB, AW), lambda i: (i, 0)),
        ],
        out_shape=[
            jax.ShapeDtypeStruct((N, hc), jnp.float32),
            jax.ShapeDtypeStruct((N, AW), jnp.float32),
        ],
    )(num, den, b, w, att_s, att_d)


def _head_body(num_ref, den_ref, b_ref, fcw_ref, fcb_ref,
               nreps_ref, grep_ref, probs_ref, acc_ref):
    i = pl.program_id(0)
    nb = _norm_block(num_ref[...], den_ref[...], b_ref[...])
    nreps_ref[...] = nb

    @pl.when(i == 0)
    def _():
        acc_ref[...] = jnp.zeros_like(acc_ref)

    acc_ref[...] += jnp.sum(nb, axis=0, keepdims=True)

    @pl.when(i == GRID - 1)
    def _():
        g = acc_ref[...] / jnp.float32(N)
        grep_ref[...] = g
        lg = jnp.dot(g, fcw_ref[...], preferred_element_type=jnp.float32)
        lg = lg + fcb_ref[...]
        m = jnp.max(lg, axis=1, keepdims=True)
        e = jnp.exp(lg - m)
        probs_ref[...] = e / jnp.sum(e, axis=1, keepdims=True)


def _head(num, den, b, fcw, fcb):
    hc = num.shape[1]
    ncls = fcw.shape[1]
    return pl.pallas_call(
        _head_body,
        grid=(GRID,),
        in_specs=[
            pl.BlockSpec((ROWB, hc), lambda i: (i, 0)),
            pl.BlockSpec((ROWB, 2 * H), lambda i: (i, 0)),
            pl.BlockSpec((1, hc), lambda i: (0, 0)),
            pl.BlockSpec((hc, ncls), lambda i: (0, 0)),
            pl.BlockSpec((1, ncls), lambda i: (0, 0)),
        ],
        out_specs=[
            pl.BlockSpec((ROWB, hc), lambda i: (i, 0)),
            pl.BlockSpec((1, hc), lambda i: (0, 0)),
            pl.BlockSpec((1, ncls), lambda i: (0, 0)),
        ],
        out_shape=[
            jax.ShapeDtypeStruct((N, hc), jnp.float32),
            jax.ShapeDtypeStruct((1, hc), jnp.float32),
            jax.ShapeDtypeStruct((1, ncls), jnp.float32),
        ],
        scratch_shapes=[pltpu.VMEM((1, hc), jnp.float32)],
    )(num, den, b, fcw, fcb)


G = 32
PEND = 96
SCAN = 1024
OWNB = 256
OWN1 = 56
LAST1 = N - NC * NS * OWNB - (NC * NS - 1) * OWN1


def _sc_edge_body(xt, a_nh, src, dst, num_o, den_o,
                  num_l, den_l, grows, asg, adg, aeb, srcv, dstv,
                  srcv2, dstv2, spend, dpend, sem, sem2, sem3, *, hc, ep):
    core = lax.axis_index("c")
    sub = lax.axis_index("s")
    wid = core * NS + sub
    nchunk = ep // SCAN
    cols_per_head = hc // H
    chunks_per_head = cols_per_head // LANES
    iota = lax.iota(jnp.int32, LANES)
    zf = jnp.zeros((LANES,), jnp.float32)
    zi = jnp.zeros((LANES,), jnp.int32)

    for i in range(G):
        aeb[i, :] = zf
    for g in range(PEND // LANES):
        plsc.store_scatter(spend, [iota + g * LANES], zi)
        plsc.store_scatter(dpend, [iota + g * LANES], zi)

    def compute_ae():
        for g in range(G // LANES):
            ii = iota + (g * LANES)
            for h in range(H):
                hv = jnp.full((LANES,), h, jnp.int32)
                av = plsc.load_gather(asg, [ii, hv])
                dv = plsc.load_gather(adg, [ii, jnp.full((LANES,), h + H,
                                                         jnp.int32)])
                e = jnp.exp(_leaky(av + dv, 0.2))
                plsc.store_scatter(aeb, [ii, hv], e)

    def run_pass(base, ownb, lastb):
        t0 = base + wid * ownb
        own = jnp.where(wid == NC * NS - 1, lastb, ownb)
        zmax = max(ownb, lastb)

        def zrow(r, carry):
            rf = jnp.full((LANES,), r, jnp.int32)
            for cch in range(hc // LANES):
                plsc.store_scatter(num_l, [rf, iota + cch * LANES], zf)
            plsc.store_scatter(den_l, [rf, iota], zf)
            return carry

        lax.fori_loop(0, zmax, zrow, 0)

        def accum_edge(i, carry):
            ifull = jnp.full((LANES,), i, jnp.int32)
            rr = plsc.load_gather(dpend, [ifull]) - t0
            aev = plsc.load_gather(aeb, [ifull, iota])
            plsc.addupdate_scatter(den_l, [rr, iota], aev)
            for h in range(H):
                m = plsc.load_gather(aeb, [ifull, jnp.full((LANES,), h,
                                                           jnp.int32)])
                for q in range(chunks_per_head):
                    col = iota + (h * cols_per_head + q * LANES)
                    v = plsc.load_gather(grows, [ifull, col])
                    plsc.addupdate_scatter(num_l, [rr, col], v * m)
            return carry

        def start_gather():
            pltpu.async_copy(xt.at[spend.at[pl.ds(0, G)]], grows, sem)
            pltpu.async_copy(a_nh.at[spend.at[pl.ds(0, G)]], asg, sem2)
            pltpu.async_copy(a_nh.at[dpend.at[pl.ds(0, G)]], adg, sem2)

        def wait_gather():
            pltpu.make_async_copy(xt.at[spend.at[pl.ds(0, G)]], grows,
                                  sem).wait()
            pltpu.make_async_copy(a_nh.at[spend.at[pl.ds(0, G)]], asg,
                                  sem2).wait()
            pltpu.make_async_copy(a_nh.at[dpend.at[pl.ds(0, G)]], adg,
                                  sem2).wait()

        def drain(nedges, off2):
            wait_gather()
            compute_ae()
            lax.fori_loop(0, nedges, accum_edge, 0)
            for gg in range((G + 15 + LANES - 1) // LANES):
                mvi = iota + (G + gg * LANES)
                mv = mvi < off2
                svr = plsc.load_gather(spend, [mvi])
                dvr = plsc.load_gather(dpend, [mvi])
                plsc.store_scatter(spend, [iota + gg * LANES], svr, mask=mv)
                plsc.store_scatter(dpend, [iota + gg * LANES], dvr, mask=mv)

        def group(buf, gi, carry):
            off, armed = carry
            ii = iota + gi * LANES
            sbuf, dbuf = buf
            dv = plsc.load_gather(dbuf, [ii])
            r = dv - t0
            ok = (r >= 0) & (r < own)
            cnt = plsc.all_reduce_population_count(ok)[0]

            @pl.when(cnt > 0)
            def _():
                sv = plsc.load_gather(sbuf, [ii])
                csum = plsc.cumsum(jnp.where(ok, 1, 0).astype(jnp.int32))
                pos = off + csum - 1
                plsc.store_scatter(spend, [pos], sv, mask=ok)
                plsc.store_scatter(dpend, [pos], dv, mask=ok)

            off2 = off + cnt

            fire_c = (armed == 0) & (off2 >= G)

            @pl.when(fire_c)
            def _():
                start_gather()

            acc_c = (armed == 1) & (off2 >= 2 * G)

            @pl.when(acc_c)
            def _():
                drain(G, off2)

                @pl.when(off2 - G >= G)
                def _():
                    start_gather()

            off3 = jnp.where(acc_c, off2 - G, off2)
            armed3 = jnp.where(
                acc_c, (off3 >= G).astype(jnp.int32),
                jnp.where(fire_c, 1, armed))
            return (off3, armed3)

        def start_scan(ci, sbuf, dbuf):
            b0 = ci * SCAN
            pltpu.async_copy(src.at[pl.ds(b0, SCAN)], sbuf, sem3)
            pltpu.async_copy(dst.at[pl.ds(b0, SCAN)], dbuf, sem3)

        def wait_scan(sbuf, dbuf):
            pltpu.make_async_copy(src.at[pl.ds(0, SCAN)], sbuf, sem3).wait()
            pltpu.make_async_copy(dst.at[pl.ds(0, SCAN)], dbuf, sem3).wait()

        def chunk(ci, carry):
            wait_scan(srcv, dstv)
            start_scan(2 * ci + 1, srcv2, dstv2)
            carry = lax.fori_loop(0, SCAN // LANES,
                                  functools.partial(group, (srcv, dstv)),
                                  carry)
            wait_scan(srcv2, dstv2)
            start_scan((2 * ci + 2) % nchunk, srcv, dstv)
            return lax.fori_loop(0, SCAN // LANES,
                                 functools.partial(group, (srcv2, dstv2)),
                                 carry)

        start_scan(0, srcv, dstv)
        off, armed = lax.fori_loop(0, nchunk // 2, chunk,
                                   (jnp.int32(0), jnp.int32(0)))
        wait_scan(srcv, dstv)

        @pl.when(armed == 1)
        def _():
            drain(G, off)

        offt = jnp.where(armed == 1, off - G, off)

        @pl.when(offt > 0)
        def _():
            start_gather()
            wait_gather()
            compute_ae()
            lax.fori_loop(0, offt, accum_edge, 0)

        @pl.when(wid < NC * NS - 1)
        def _():
            pltpu.sync_copy(num_l.at[pl.ds(0, ownb)],
                            num_o.at[pl.ds(t0, ownb)])
            pltpu.sync_copy(den_l.at[pl.ds(0, ownb)],
                            den_o.at[pl.ds(t0, ownb)])

        @pl.when(wid == NC * NS - 1)
        def _():
            pltpu.sync_copy(num_l.at[pl.ds(0, lastb)],
                            num_o.at[pl.ds(t0, lastb)])
            pltpu.sync_copy(den_l.at[pl.ds(0, lastb)],
                            den_o.at[pl.ds(t0, lastb)])

    run_pass(0, OWNB, OWNB)
    run_pass(NC * NS * OWNB, OWN1, LAST1)


def _sc_edge(xt, a_nh, src, dst, hc, ep):
    mesh = plsc.VectorSubcoreMesh(core_axis_name="c", subcore_axis_name="s",
                                  num_cores=NC, num_subcores=NS)
    kfn = pl.kernel(
        functools.partial(_sc_edge_body, hc=hc, ep=ep),
        out_type=(
            jax.ShapeDtypeStruct((N, hc), jnp.float32),
            jax.ShapeDtypeStruct((N, 2 * H), jnp.float32),
        ),
        mesh=mesh,
        scratch_types=[
            pltpu.VMEM((OWNB, hc), jnp.float32),
            pltpu.VMEM((OWNB, 2 * H), jnp.float32),
            pltpu.VMEM((G, hc), jnp.float32),
            pltpu.VMEM((G, AW), jnp.float32),
            pltpu.VMEM((G, AW), jnp.float32),
            pltpu.VMEM((G, 2 * H), jnp.float32),
            pltpu.VMEM((SCAN,), jnp.int32),
            pltpu.VMEM((SCAN,), jnp.int32),
            pltpu.VMEM((SCAN,), jnp.int32),
            pltpu.VMEM((SCAN,), jnp.int32),
            pltpu.VMEM((PEND,), jnp.int32),
            pltpu.VMEM((PEND,), jnp.int32),
            pltpu.SemaphoreType.DMA,
            pltpu.SemaphoreType.DMA,
            pltpu.SemaphoreType.DMA,
        ],
        compiler_params=pltpu.CompilerParams(needs_layout_passes=False),
    )
    return kfn(xt, a_nh, src, dst)


def kernel(x, edge_index, W1, att_src1, att_dst1, b1,
           W2, att_src2, att_dst2, b2, fcW, fcb):
    e = edge_index.shape[1]
    etot = e + N
    ep = ((etot + 2 * SCAN - 1) // (2 * SCAN)) * (2 * SCAN)
    pad = ep - etot
    loop = jnp.arange(N, dtype=edge_index.dtype)
    src = jnp.concatenate(
        [edge_index[0], loop, jnp.zeros((pad,), edge_index.dtype)])
    dst = jnp.concatenate(
        [edge_index[1], loop, jnp.full((pad,), -1, edge_index.dtype)])

    xt1, a1 = _proj(x, W1, att_src1, att_dst1)
    num1, den1 = _sc_edge(xt1, a1, src, dst, W1.shape[1], ep)
    xt2, a2 = _proj2(num1, den1, b1.reshape(1, -1), W2, att_src2, att_dst2)
    num2, den2 = _sc_edge(xt2, a2, src, dst, W2.shape[1], ep)
    n_reps, g_rep, probs = _head(num2, den2, b2.reshape(1, -1),
                                 fcW, fcb.reshape(1, -1))
    return (probs, n_reps, g_rep)
